# baseline (speedup 1.0000x reference)
import functools

import jax
import jax.numpy as jnp
from jax import lax
from jax.experimental import pallas as pl
from jax.experimental.pallas import tpu as pltpu
from jax.experimental.pallas import tpu_sc as plsc

MASK_LO = 900000
DIM = 64
CHUNK = 160
NBUF = 4


def _popcount16(x):
    x = x - ((x >> 1) & 0x5555)
    x = (x & 0x3333) + ((x >> 2) & 0x3333)
    x = (x + (x >> 4)) & 0x0F0F
    return (x + (x >> 8)) & 0x1F


def _sc_embed(idx, W_main, W_mask):
    N = idx.shape[0]
    info = plsc.get_sparse_core_info()
    NC, NS, L = info.num_cores, info.num_subcores, info.num_lanes
    NW = NC * NS
    assert N % (NW * CHUNK * NBUF) == 0
    per_w = N // NW
    n_chunks = per_w // CHUNK
    n_vecs = CHUNK // L

    mesh = plsc.VectorSubcoreMesh(core_axis_name="c", subcore_axis_name="s")

    scratch = [pltpu.VMEM((per_w,), jnp.int32)]
    scratch += [pltpu.VMEM((CHUNK,), jnp.int32) for _ in range(NBUF)]
    scratch += [pltpu.VMEM((CHUNK, DIM), jnp.float32) for _ in range(NBUF)]
    scratch += [pltpu.VMEM((CHUNK, DIM), jnp.float32) for _ in range(NBUF)]
    scratch += [pltpu.SemaphoreType.DMA] * (3 * NBUF + 1)

    @functools.partial(
        pl.kernel,
        out_type=jax.ShapeDtypeStruct((N, DIM), jnp.float32),
        mesh=mesh,
        scratch_types=scratch,
        compiler_params=pltpu.CompilerParams(use_tc_tiling_on_sc=False),
    )
    def k(idx_hbm, wmain_hbm, wmask_hbm, out_hbm, slab, *rest):
        midx = rest[0:NBUF]
        rows = rest[NBUF:2 * NBUF]
        mrows = rest[2 * NBUF:3 * NBUF]
        s_gm = rest[3 * NBUF:4 * NBUF]
        s_gk = rest[4 * NBUF:5 * NBUF]
        s_w = rest[5 * NBUF:6 * NBUF]
        s_slab = rest[6 * NBUF]

        wid = lax.axis_index("s") * NC + lax.axis_index("c")
        wbase = wid * per_w
        lane = lax.iota(jnp.int32, L)
        rots = [((lane + sh) & (L - 1)).astype(jnp.int32) for sh in (8, 4, 2, 1)]

        pltpu.async_copy(idx_hbm.at[pl.ds(wbase, per_w)], slab, s_slab).wait()

        def issue(c, j):
            def vbody(v, _):
                a = slab[pl.ds(c * CHUNK + v * L, L)]
                midx[j][pl.ds(v * L, L)] = jnp.maximum(a - MASK_LO, 0)
                return 0

            lax.fori_loop(0, n_vecs, vbody, 0, unroll=2)
            pltpu.async_copy(
                wmain_hbm.at[slab.at[pl.ds(c * CHUNK, CHUNK)]], rows[j], s_gm[j])
            pltpu.async_copy(wmask_hbm.at[midx[j]], mrows[j], s_gk[j])

        def wait_write(c, j):
            pltpu.make_async_copy(
                rows[j], out_hbm.at[pl.ds(wbase + c * CHUNK, CHUNK)],
                s_w[j]).wait()

        def finish(c, j):
            pltpu.make_async_copy(
                wmain_hbm.at[slab.at[pl.ds(c * CHUNK, CHUNK)]], rows[j],
                s_gm[j]).wait()
            pltpu.make_async_copy(
                wmask_hbm.at[midx[j]], mrows[j], s_gk[j]).wait()

            def vbody(v, _):
                a = slab[pl.ds(c * CHUNK + v * L, L)]
                bits = jnp.where(a >= MASK_LO, jnp.int32(1) << lane,
                                 jnp.int32(0))
                for r in rots:
                    bits = bits | bits.at[r].get(mode="promise_in_bounds")
                mb = bits[0]
                nmv = _popcount16(mb)

                def tbody(_, carry):
                    low = carry & (-carry)
                    f = low.astype(jnp.float32)
                    t = v * L + ((lax.bitcast_convert_type(f, jnp.int32)
                                  >> 23) - 127)
                    for kk in range(DIM // L):
                        rows[j].at[t][pl.ds(kk * L, L)] = (
                            mrows[j].at[t][pl.ds(kk * L, L)])
                    return carry & (carry - 1)

                lax.fori_loop(0, nmv, tbody, mb)
                return 0

            lax.fori_loop(0, n_vecs, vbody, 0)
            pltpu.async_copy(
                rows[j], out_hbm.at[pl.ds(wbase + c * CHUNK, CHUNK)], s_w[j])

        def step(i, _):
            for j in range(NBUF):
                c = NBUF * i + j
                if j == 0:
                    @pl.when(i >= 1)
                    def _():
                        wait_write(c - NBUF, 0)
                        issue(c, 0)
                        finish(c - 1, NBUF - 1)

                    @pl.when(i == 0)
                    def _():
                        issue(c, 0)
                else:
                    @pl.when(i >= 1)
                    def _():
                        wait_write(c - NBUF, j)

                    issue(c, j)
                    finish(c - 1, j - 1)
            return 0

        lax.fori_loop(0, n_chunks // NBUF, step, 0)
        finish(n_chunks - 1, NBUF - 1)
        for j in range(NBUF):
            wait_write(n_chunks - NBUF + j, j)

    return k(idx, W_main, W_mask)


def kernel(input, W_main, W_mask):
    B, H = input.shape
    out = _sc_embed(input.reshape(B * H), W_main, W_mask)
    return out.reshape(B, H, DIM)


# speedup vs baseline: 7.1096x; 7.1096x over previous
import functools

import jax
import jax.numpy as jnp
from jax import lax
from jax.experimental import pallas as pl
from jax.experimental.pallas import tpu as pltpu
from jax.experimental.pallas import tpu_sc as plsc

MASK_LO = 900000
DIM = 64
CHUNK = 160
NBUF = 4


def _sc_embed(idx, W_main, W_mask):
    N = idx.shape[0]
    info = plsc.get_sparse_core_info()
    NC, NS, L = info.num_cores, info.num_subcores, info.num_lanes
    NW = NC * NS
    assert N % (NW * CHUNK * NBUF) == 0
    per_w = N // NW
    n_chunks = per_w // CHUNK
    n_vecs = CHUNK // L

    mesh = plsc.VectorSubcoreMesh(core_axis_name="c", subcore_axis_name="s")

    scratch = [pltpu.VMEM((per_w,), jnp.int32)]
    scratch += [pltpu.VMEM((CHUNK,), jnp.int32) for _ in range(NBUF)]
    scratch += [pltpu.VMEM((CHUNK,), jnp.int32) for _ in range(NBUF)]
    scratch += [pltpu.VMEM((CHUNK,), jnp.int32) for _ in range(NBUF)]
    scratch += [pltpu.VMEM((CHUNK,), jnp.int32) for _ in range(NBUF)]
    scratch += [pltpu.VMEM((CHUNK, DIM), jnp.float32) for _ in range(NBUF)]
    scratch += [pltpu.VMEM((CHUNK, DIM), jnp.float32) for _ in range(NBUF)]
    scratch += [pltpu.SMEM((2 * NBUF,), jnp.int32)]
    scratch += [pltpu.SemaphoreType.DMA] * (4 * NBUF + 1)

    @functools.partial(
        pl.kernel,
        out_type=jax.ShapeDtypeStruct((N, DIM), jnp.float32),
        mesh=mesh,
        scratch_types=scratch,
        compiler_params=pltpu.CompilerParams(use_tc_tiling_on_sc=False),
    )
    def k(idx_hbm, wmain_hbm, wmask_hbm, out_hbm, slab, *rest):
        idxm = rest[0:NBUF]
        midx = rest[NBUF:2 * NBUF]
        gposA = rest[2 * NBUF:3 * NBUF]
        gposB = rest[3 * NBUF:4 * NBUF]
        rows = rest[4 * NBUF:5 * NBUF]
        mrows = rest[5 * NBUF:6 * NBUF]
        fl = rest[6 * NBUF]
        s_gm = rest[6 * NBUF + 1:7 * NBUF + 1]
        s_gk = rest[7 * NBUF + 1:8 * NBUF + 1]
        s_sa = rest[8 * NBUF + 1:9 * NBUF + 1]
        s_sb = rest[9 * NBUF + 1:10 * NBUF + 1]
        s_slab = rest[10 * NBUF + 1]

        wid = lax.axis_index("s") * NC + lax.axis_index("c")
        wbase = wid * per_w
        lane = lax.iota(jnp.int32, L)
        rots = [((lane + sh) & (L - 1)).astype(jnp.int32) for sh in (8, 4, 2, 1)]

        pltpu.async_copy(idx_hbm.at[pl.ds(wbase, per_w)], slab, s_slab).wait()

        def wait_sa(j):
            pltpu.make_async_copy(rows[j], out_hbm.at[gposA[j]], s_sa[j]).wait()

        def wait_sb(j):
            pltpu.make_async_copy(mrows[j], out_hbm.at[gposB[j]], s_sb[j]).wait()

        def issue(c, j):
            cbase = wbase + c * CHUNK

            def scan_vec(v, carry):
                mm, mn = carry
                a = slab[pl.ds(c * CHUNK + v * L, L)]
                is_m = a >= MASK_LO
                pos = v * L + lane
                keym = jnp.where(is_m, (pos << 17) | (a - MASK_LO), -1)
                keyn = jnp.where(is_m, -1, (pos << 20) | a)
                return (jnp.maximum(mm, keym), jnp.maximum(mn, keyn))

            init = jnp.full((L,), -1, jnp.int32)
            Mm, Mn = lax.fori_loop(0, n_vecs, scan_vec, (init, init),
                                   unroll=2)
            for r in rots:
                Mm = jnp.maximum(Mm, Mm.at[r].get(mode="promise_in_bounds"))
                Mn = jnp.maximum(Mn, Mn.at[r].get(mode="promise_in_bounds"))
            fl[j] = (Mm[0] >= 0).astype(jnp.int32)
            fl[NBUF + j] = (Mn[0] >= 0).astype(jnp.int32)
            Mmc = jnp.maximum(Mm, 0)
            Mnc = jnp.maximum(Mn, 0)
            fm_pos, fm_midx = Mmc >> 17, Mmc & 0x1FFFF
            fn_pos, fn_id = Mnc >> 20, Mnc & 0xFFFFF

            def fix_vec(v, _):
                a = slab[pl.ds(c * CHUNK + v * L, L)]
                is_m = a >= MASK_LO
                pos = v * L + lane
                sl = pl.ds(v * L, L)
                idxm[j][sl] = jnp.where(is_m, fn_id, a)
                midx[j][sl] = jnp.where(is_m, a - MASK_LO, fm_midx)
                gposA[j][sl] = cbase + jnp.where(is_m, fn_pos, pos)
                gposB[j][sl] = cbase + jnp.where(is_m, pos, fm_pos)
                return 0

            lax.fori_loop(0, n_vecs, fix_vec, 0, unroll=2)
            pltpu.async_copy(wmain_hbm.at[idxm[j]], rows[j], s_gm[j])
            pltpu.async_copy(wmask_hbm.at[midx[j]], mrows[j], s_gk[j])

        def flush(c, j):
            pltpu.make_async_copy(
                wmain_hbm.at[idxm[j]], rows[j], s_gm[j]).wait()
            pltpu.make_async_copy(
                wmask_hbm.at[midx[j]], mrows[j], s_gk[j]).wait()

            @pl.when(fl[NBUF + j] != 0)
            def _():
                pltpu.async_copy(rows[j], out_hbm.at[gposA[j]], s_sa[j])

            @pl.when(fl[j] != 0)
            def _():
                pltpu.async_copy(mrows[j], out_hbm.at[gposB[j]], s_sb[j])

        def step(i, _):
            for j in range(NBUF):
                c = NBUF * i + j
                jw = (j - 1) % NBUF

                @pl.when(i >= 1)
                def _():
                    @pl.when(fl[NBUF + j] != 0)
                    def _():
                        wait_sa(j)

                    @pl.when(fl[j] != 0)
                    def _():
                        wait_sb(j)

                    issue(c, j)
                    flush(c - 1, jw)

                @pl.when(i == 0)
                def _():
                    issue(j, j)
                    if j >= 1:
                        flush(j - 1, j - 1)
            return 0

        lax.fori_loop(0, n_chunks // NBUF, step, 0)
        flush(n_chunks - 1, (n_chunks - 1) % NBUF)
        for cc in range(n_chunks - NBUF, n_chunks):
            @pl.when(fl[NBUF + cc % NBUF] != 0)
            def _(cc=cc):
                wait_sa(cc % NBUF)

            @pl.when(fl[cc % NBUF] != 0)
            def _(cc=cc):
                wait_sb(cc % NBUF)

    return k(idx, W_main, W_mask)


def kernel(input, W_main, W_mask):
    B, H = input.shape
    out = _sc_embed(input.reshape(B * H), W_main, W_mask)
    return out.reshape(B, H, DIM)


# speedup vs baseline: 7.4040x; 1.0414x over previous
import functools

import jax
import jax.numpy as jnp
from jax import lax
from jax.experimental import pallas as pl
from jax.experimental.pallas import tpu as pltpu
from jax.experimental.pallas import tpu_sc as plsc

MASK_LO = 900000
DIM = 64
CHUNK = 160
NBUF = 4


def _sc_embed(idx, W_main, W_mask):
    N = idx.shape[0]
    info = plsc.get_sparse_core_info()
    NC, NS, L = info.num_cores, info.num_subcores, info.num_lanes
    NW = NC * NS
    assert N % (NW * CHUNK * NBUF) == 0
    per_w = N // NW
    n_chunks = per_w // CHUNK
    n_vecs = CHUNK // L

    mesh = plsc.VectorSubcoreMesh(core_axis_name="c", subcore_axis_name="s")

    scratch = [pltpu.VMEM((per_w,), jnp.int32)]
    scratch += [pltpu.VMEM((CHUNK,), jnp.int32) for _ in range(NBUF)]
    scratch += [pltpu.VMEM((CHUNK,), jnp.int32) for _ in range(NBUF)]
    scratch += [pltpu.VMEM((CHUNK,), jnp.int32) for _ in range(NBUF)]
    scratch += [pltpu.VMEM((CHUNK,), jnp.int32) for _ in range(NBUF)]
    scratch += [pltpu.VMEM((CHUNK, DIM), jnp.float32) for _ in range(NBUF)]
    scratch += [pltpu.VMEM((CHUNK, DIM), jnp.float32) for _ in range(NBUF)]
    scratch += [pltpu.SMEM((2 * NBUF,), jnp.int32)]
    scratch += [pltpu.SemaphoreType.DMA] * (4 * NBUF + 1)

    @functools.partial(
        pl.kernel,
        out_type=jax.ShapeDtypeStruct((N, DIM), jnp.float32),
        mesh=mesh,
        scratch_types=scratch,
        compiler_params=pltpu.CompilerParams(use_tc_tiling_on_sc=False),
    )
    def k(idx_hbm, wmain_hbm, wmask_hbm, out_hbm, slab, *rest):
        idxm = rest[0:NBUF]
        midx = rest[NBUF:2 * NBUF]
        gposA = rest[2 * NBUF:3 * NBUF]
        gposB = rest[3 * NBUF:4 * NBUF]
        rows = rest[4 * NBUF:5 * NBUF]
        mrows = rest[5 * NBUF:6 * NBUF]
        fl = rest[6 * NBUF]
        s_gm = rest[6 * NBUF + 1:7 * NBUF + 1]
        s_gk = rest[7 * NBUF + 1:8 * NBUF + 1]
        s_sa = rest[8 * NBUF + 1:9 * NBUF + 1]
        s_sb = rest[9 * NBUF + 1:10 * NBUF + 1]
        s_slab = rest[10 * NBUF + 1]

        wid = lax.axis_index("s") * NC + lax.axis_index("c")
        wbase = wid * per_w
        lane = lax.iota(jnp.int32, L)
        rots = [((lane + sh) & (L - 1)).astype(jnp.int32) for sh in (8, 4, 2, 1)]

        pltpu.async_copy(idx_hbm.at[pl.ds(wbase, per_w)], slab, s_slab).wait()

        def wait_sa(j):
            pltpu.make_async_copy(rows[j], out_hbm.at[gposA[j]], s_sa[j]).wait()

        def wait_sb(j):
            pltpu.make_async_copy(mrows[j], out_hbm.at[gposB[j]], s_sb[j]).wait()

        def issue(c, j):
            cbase = wbase + c * CHUNK

            def scan_vec(v, carry):
                mm, mn = carry
                a = slab[pl.ds(c * CHUNK + v * L, L)]
                is_m = a >= MASK_LO
                pos = v * L + lane
                keym = jnp.where(is_m, (pos << 17) | (a - MASK_LO), -1)
                keyn = jnp.where(is_m, -1, (pos << 20) | a)
                return (jnp.maximum(mm, keym), jnp.maximum(mn, keyn))

            init = jnp.full((L,), -1, jnp.int32)
            Mm, Mn = lax.fori_loop(0, n_vecs, scan_vec, (init, init),
                                   unroll=2)
            for r in rots:
                Mm = jnp.maximum(Mm, Mm.at[r].get(mode="promise_in_bounds"))
                Mn = jnp.maximum(Mn, Mn.at[r].get(mode="promise_in_bounds"))
            fl[j] = (Mm[0] >= 0).astype(jnp.int32)
            fl[NBUF + j] = (Mn[0] >= 0).astype(jnp.int32)
            Mmc = jnp.maximum(Mm, 0)
            Mnc = jnp.maximum(Mn, 0)
            fm_pos, fm_midx = Mmc >> 17, Mmc & 0x1FFFF
            fn_pos, fn_id = Mnc >> 20, Mnc & 0xFFFFF

            def fix_vec(v, _):
                a = slab[pl.ds(c * CHUNK + v * L, L)]
                is_m = a >= MASK_LO
                pos = v * L + lane
                sl = pl.ds(v * L, L)
                idxm[j][sl] = jnp.where(is_m, fn_id, a)
                midx[j][sl] = jnp.where(is_m, a - MASK_LO, fm_midx)
                gposA[j][sl] = cbase + jnp.where(is_m, fn_pos, pos)
                gposB[j][sl] = cbase + jnp.where(is_m, pos, fm_pos)
                return 0

            lax.fori_loop(0, n_vecs, fix_vec, 0, unroll=2)
            pltpu.async_copy(wmain_hbm.at[idxm[j]], rows[j], s_gm[j])
            pltpu.async_copy(wmask_hbm.at[midx[j]], mrows[j], s_gk[j])

        def flush(c, j):
            pltpu.make_async_copy(
                wmain_hbm.at[idxm[j]], rows[j], s_gm[j]).wait()
            pltpu.make_async_copy(
                wmask_hbm.at[midx[j]], mrows[j], s_gk[j]).wait()

            @pl.when(fl[NBUF + j] != 0)
            def _():
                pltpu.async_copy(rows[j], out_hbm.at[gposA[j]], s_sa[j])

            @pl.when(fl[j] != 0)
            def _():
                pltpu.async_copy(mrows[j], out_hbm.at[gposB[j]], s_sb[j])

        def step(i, _):
            for j in range(NBUF):
                c = NBUF * i + j
                jw = (j - 2) % NBUF

                @pl.when(i >= 1)
                def _():
                    @pl.when(fl[NBUF + j] != 0)
                    def _():
                        wait_sa(j)

                    @pl.when(fl[j] != 0)
                    def _():
                        wait_sb(j)

                    issue(c, j)
                    flush(c - 2, jw)

                @pl.when(i == 0)
                def _():
                    issue(j, j)
                    if j >= 2:
                        flush(j - 2, j - 2)
            return 0

        lax.fori_loop(0, n_chunks // NBUF, step, 0)
        flush(n_chunks - 2, (n_chunks - 2) % NBUF)
        flush(n_chunks - 1, (n_chunks - 1) % NBUF)
        for cc in range(n_chunks - NBUF, n_chunks):
            @pl.when(fl[NBUF + cc % NBUF] != 0)
            def _(cc=cc):
                wait_sa(cc % NBUF)

            @pl.when(fl[cc % NBUF] != 0)
            def _(cc=cc):
                wait_sb(cc % NBUF)

    return k(idx, W_main, W_mask)


def kernel(input, W_main, W_mask):
    B, H = input.shape
    out = _sc_embed(input.reshape(B * H), W_main, W_mask)
    return out.reshape(B, H, DIM)
